# baseline (speedup 1.0000x reference)
import functools

import jax
import jax.numpy as jnp
from jax import lax
from jax.experimental import pallas as pl
from jax.experimental.pallas import tpu as pltpu
from jax.experimental.pallas import tpu_sc as plsc


def _make_gather(B, V, D):
    info = plsc.get_sparse_core_info()
    NC, NS = info.num_cores, info.num_subcores
    NW = NC * NS
    assert B % NW == 0
    b_per_w = B // NW
    NSEM = 8
    UNROLL = 8

    mesh = plsc.VectorSubcoreMesh(core_axis_name="c", subcore_axis_name="s")

    @functools.partial(
        pl.kernel,
        mesh=mesh,
        out_type=jax.ShapeDtypeStruct((B, D), jnp.float32),
        scratch_types=[
            pltpu.VMEM_SHARED((NW, b_per_w), jnp.int32),
            pltpu.SMEM((b_per_w,), jnp.int32),
            pltpu.VMEM((b_per_w, D), jnp.float32),
            [pltpu.SemaphoreType.DMA] * NSEM,
        ],
    )
    def k(ids_hbm, table_hbm, out_hbm, idx_sh, idx_s, rows_v, sems):
        wid = lax.axis_index("s") * NC + lax.axis_index("c")
        base = wid * b_per_w
        pltpu.sync_copy(ids_hbm.at[pl.ds(base, b_per_w)], idx_sh.at[wid])
        pltpu.sync_copy(idx_sh.at[wid], idx_s)

        def body(t, carry):
            j0 = t * UNROLL
            for u in range(UNROLL):
                j = j0 + u
                i = idx_s[j]
                pltpu.async_copy(
                    table_hbm.at[pl.ds(i, 1)],
                    rows_v.at[pl.ds(j, 1)],
                    sems[u % NSEM],
                )
            return carry

        lax.fori_loop(0, b_per_w // UNROLL, body, 0)
        per_sem = b_per_w // NSEM
        for u in range(NSEM):
            pltpu.make_async_copy(
                table_hbm.at[pl.ds(0, per_sem)],
                rows_v.at[pl.ds(u * per_sem, per_sem)],
                sems[u],
            ).wait()
        pltpu.sync_copy(rows_v, out_hbm.at[pl.ds(base, b_per_w)])

    return k


def kernel(style_ids, table):
    (B,) = style_ids.shape
    V, D = table.shape
    gather = _make_gather(B, V, D)
    return gather(style_ids.astype(jnp.int32), table)
